# baseline (speedup 1.0000x reference)
import functools

import jax
import jax.numpy as jnp
from jax import lax
from jax.experimental import pallas as pl
from jax.experimental.pallas import tpu as pltpu
from jax.experimental.pallas import tpu_sc as plsc

_N = 10000
_E = 320000
_D_IN = 128
_D_H = 256
_D_OUT = 2

_C1 = 48
_NS1 = 12
_R1 = 36
_CHUNK = 128
_K2 = 80
_E_PAD1 = 16 * _R1 * _NS1 * _C1
_E_PAD2 = 32 * _K2 * _CHUNK
_N2 = 10240
_W1 = 64
_W2 = 16
_RPS = _N2 // 16
_NSLOT = 8

_mesh = plsc.VectorSubcoreMesh(core_axis_name="c", subcore_axis_name="s")
_sc_params = pltpu.CompilerParams(use_tc_tiling_on_sc=False)


def _fill_rows(buf, value):
    n, w = buf.shape
    vec = jnp.full((16,), value, jnp.float32)

    def row(i, carry):
        for j in range(w // 16):
            buf[i, pl.ds(j * 16, 16)] = vec
        return carry

    lax.fori_loop(0, n, row, 0)


def _blast_stripe(buf, acc_sh, s):
    n = buf.shape[0]
    for t in range(_RPS // n):
        pltpu.sync_copy(buf, acc_sh.at[pl.ds(s * _RPS + t * n, n)])


def _edge_rounds(table, src_v, dst_v, bufs, acc_sh, sem_g, sem_s, nchunks,
                 cnt_cb=None, cnt_drain=None):
    nrounds = nchunks // _NSLOT

    for j in range(_NSLOT):
        pltpu.async_copy(table.at[src_v.at[j]], bufs[j], sem_g)

    def body(i, carry):
        k0 = i * _NSLOT
        for j in range(_NSLOT):
            k = k0 + j
            pltpu.make_async_copy(table.at[src_v.at[k]], bufs[j], sem_g).wait()
            pltpu.async_copy(bufs[j], acc_sh.at[dst_v.at[k]], sem_s, add=True)
            if cnt_cb is not None:
                cnt_cb(j, k)
        for j in range(_NSLOT):
            pltpu.make_async_copy(
                bufs[j], acc_sh.at[dst_v.at[k0 + j]], sem_s).wait()
        if cnt_drain is not None:
            cnt_drain(k0)

        @pl.when(i < nrounds - 1)
        def _():
            for j in range(_NSLOT):
                pltpu.async_copy(
                    table.at[src_v.at[k0 + _NSLOT + j]], bufs[j], sem_g)

        return carry

    lax.fori_loop(0, nrounds, body, 0)


@functools.partial(
    pl.kernel,
    mesh=_mesh,
    compiler_params=_sc_params,
    out_type=(
        jax.ShapeDtypeStruct((_N2, _W1), jnp.float32),
        jax.ShapeDtypeStruct((_N2, _W1), jnp.float32),
        jax.ShapeDtypeStruct((2, _N2, _W2), jnp.float32),
    ),
    scratch_types=[
        [pltpu.VMEM((2, _NS1, _C1), jnp.int32)] * 2,
        [pltpu.VMEM((_C1, _W1), jnp.float32)] * (2 * _NS1),
        pltpu.VMEM((_C1, _W2), jnp.float32),
        pltpu.VMEM((64, _W2), jnp.float32),
        pltpu.VMEM_SHARED((_N2, _W1), jnp.float32),
        pltpu.VMEM_SHARED((_N2, _W2), jnp.float32),
        [pltpu.SemaphoreType.DMA] * 2,
        [pltpu.SemaphoreType.DMA] * 2,
        pltpu.SemaphoreType.DMA,
        pltpu.SemaphoreType.DMA,
    ],
)
def _sc_pass1(t0_hbm, t1_hbm, idx_hbm, out_lo, out_hi, out_cnt,
              ibanks, bufs_flat, ones_v, zeros_v, acc_sh, cnt_sh,
              sem_i, sem_g, sem_s, sem_c):
    c = lax.axis_index("c")
    s = lax.axis_index("s")
    bufs = [bufs_flat[:_NS1], bufs_flat[_NS1:]]

    _fill_rows(bufs[0][0], 0.0)
    _fill_rows(ones_v, 1.0)
    _fill_rows(zeros_v, 0.0)
    _blast_stripe(bufs[0][0].at[pl.ds(0, 32)], acc_sh, s)
    _blast_stripe(zeros_v, cnt_sh, s)
    plsc.subcore_barrier()

    def run(table):
        pltpu.sync_copy(idx_hbm.at[s, 0], ibanks[0])
        for j in range(_NS1):
            pltpu.async_copy(
                table.at[ibanks[0].at[0, j]], bufs[0][j], sem_g[0])
        pltpu.async_copy(idx_hbm.at[s, 1], ibanks[1], sem_i[1])

        def half_round(r, x):
            y = 1 - x

            @pl.when(r + 1 < _R1)
            def _():
                pltpu.make_async_copy(
                    idx_hbm.at[s, 0], ibanks[y], sem_i[y]).wait()
                for j in range(_NS1):
                    pltpu.async_copy(
                        table.at[ibanks[y].at[0, j]], bufs[y][j], sem_g[y])

            for j in range(_NS1):
                pltpu.make_async_copy(
                    table.at[ibanks[x].at[0, j]], bufs[x][j], sem_g[x]).wait()
                pltpu.async_copy(
                    bufs[x][j], acc_sh.at[ibanks[x].at[1, j]], sem_s, add=True)

                @pl.when(c == j % 2)
                def _():
                    pltpu.async_copy(
                        ones_v, cnt_sh.at[ibanks[x].at[1, j]], sem_c, add=True)

            for j in range(_NS1):
                pltpu.make_async_copy(
                    bufs[x][j], acc_sh.at[ibanks[x].at[1, j]], sem_s).wait()
            for _unused in range(_NS1 // 2):
                pltpu.make_async_copy(
                    ones_v, cnt_sh.at[ibanks[x].at[1, 0]], sem_c).wait()

            @pl.when(r + 2 < _R1)
            def _():
                pltpu.async_copy(idx_hbm.at[s, r + 2], ibanks[x], sem_i[x])

        def body(i, carry):
            half_round(2 * i, 0)
            half_round(2 * i + 1, 1)
            return carry

        lax.fori_loop(0, _R1 // 2, body, 0)

    @pl.when(c == 0)
    def _():
        run(t0_hbm)

    @pl.when(c == 1)
    def _():
        run(t1_hbm)

    plsc.subcore_barrier()

    rows = pl.ds(s * _RPS, _RPS)

    @pl.when(c == 0)
    def _():
        pltpu.sync_copy(acc_sh.at[rows], out_lo.at[rows])

    @pl.when(c == 1)
    def _():
        pltpu.sync_copy(acc_sh.at[rows], out_hi.at[rows])

    pltpu.sync_copy(cnt_sh.at[rows], out_cnt.at[c, rows])


@functools.partial(
    pl.kernel,
    mesh=_mesh,
    compiler_params=_sc_params,
    out_type=jax.ShapeDtypeStruct((2, _N2, _W2), jnp.float32),
    scratch_types=[
        pltpu.VMEM((_K2, _CHUNK), jnp.int32),
        pltpu.VMEM((_K2, _CHUNK), jnp.int32),
        [pltpu.VMEM((_CHUNK, _W2), jnp.float32)] * _NSLOT,
        pltpu.VMEM_SHARED((_N2, _W2), jnp.float32),
        pltpu.SemaphoreType.DMA,
        pltpu.SemaphoreType.DMA,
    ],
)
def _sc_pass2(q_hbm, src_hbm, dst_hbm, out_hbm,
              src_v, dst_v, bufs, acc_sh, sem_g, sem_s):
    c = lax.axis_index("c")
    s = lax.axis_index("s")
    wid = s * 2 + c

    pltpu.sync_copy(src_hbm.at[pl.ds(wid * _K2, _K2)], src_v)
    pltpu.sync_copy(dst_hbm.at[pl.ds(wid * _K2, _K2)], dst_v)
    _fill_rows(bufs[0], 0.0)
    _blast_stripe(bufs[0], acc_sh, s)
    plsc.subcore_barrier()

    _edge_rounds(q_hbm, src_v, dst_v, bufs, acc_sh, sem_g, sem_s, _K2)

    plsc.subcore_barrier()
    rows = pl.ds(s * _RPS, _RPS)
    pltpu.sync_copy(acc_sh.at[rows], out_hbm.at[c, rows])


_R = 640


def _dense_body(lo_ref, hi_ref, cnt_ref, x_ref, wlo_ref, whi_ref, wrst_ref,
                b1_ref, gamma_ref, beta_ref, w2_ref, b2_ref,
                q0_ref, r_ref, invc_ref):
    cnt = cnt_ref[0, :, 0:1] + cnt_ref[1, :, 0:1]
    inv = 1.0 / jnp.maximum(cnt, 1.0)
    x1 = (jnp.dot(lo_ref[...] * inv, wlo_ref[...],
                  preferred_element_type=jnp.float32)
          + jnp.dot(hi_ref[...] * inv, whi_ref[...],
                    preferred_element_type=jnp.float32)
          + jnp.dot(x_ref[...], wrst_ref[...],
                    preferred_element_type=jnp.float32)
          + b1_ref[...])
    mu = jnp.mean(x1, axis=-1, keepdims=True)
    var = jnp.mean((x1 - mu) ** 2, axis=-1, keepdims=True)
    xn = (x1 - mu) * lax.rsqrt(var + 1e-5) * gamma_ref[...] + beta_ref[...]
    h = jnp.where(xn > 0, xn, jnp.exp(xn) - 1.0)
    qr = jnp.dot(h, w2_ref[...], preferred_element_type=jnp.float32) \
        + b2_ref[...]
    q0_ref[...] = qr[:, :_W2]
    r_ref[...] = qr[:, _W2:]
    invc_ref[...] = jnp.broadcast_to(inv, (_R, _W2))


def _combine_body(acc2_ref, invc_ref, r_ref, out_ref):
    out_ref[...] = (acc2_ref[0] + acc2_ref[1]) * invc_ref[...] + r_ref[...]


def kernel(x, edge_index, Wl1, bl1, Wr1, Ws, bs, gamma, beta, Wl2, bl2, Wr2):
    src = edge_index[0]
    dst = edge_index[1]
    pad1 = _E_PAD1 - _E
    src_p1 = jnp.concatenate([src, jnp.zeros((pad1,), jnp.int32)])
    dst_p1 = jnp.concatenate([dst, jnp.full((pad1,), _N, jnp.int32)])
    pad2 = _E_PAD2 - _E
    src_p2 = jnp.concatenate([src, jnp.zeros((pad2,), jnp.int32)])
    dst_p2 = jnp.concatenate([dst, jnp.full((pad2,), _N, jnp.int32)])
    src2d = src_p2.reshape(32 * _K2, _CHUNK)
    dst2d = dst_p2.reshape(32 * _K2, _CHUNK)
    src4 = src_p1.reshape(16, _R1, _NS1, _C1)
    dst4 = dst_p1.reshape(16, _R1, _NS1, _C1)
    idx_blk = jnp.stack([src4, dst4], axis=2)

    t0 = jnp.zeros((_N2, _W1), jnp.float32).at[:_N].set(x[:, :64])
    t1 = jnp.zeros((_N2, _W1), jnp.float32).at[:_N].set(x[:, 64:128])
    x_pad = jnp.zeros((_N2, _D_IN), jnp.float32).at[:_N].set(x)

    acc_lo, acc_hi, cnt2 = _sc_pass1(t0, t1, idx_blk)

    wl1t = Wl1.T
    wlo = wl1t[:64]
    whi = wl1t[64:128]
    wrst = (Wr1 + Ws).T
    b1 = (bl1 + bs).reshape(1, _D_H)
    g2 = gamma.reshape(1, _D_H)
    be2 = beta.reshape(1, _D_H)
    w2 = jnp.zeros((_D_H, 2 * _W2), jnp.float32)
    w2 = w2.at[:, 0:_D_OUT].set(Wl2.T).at[:, _W2:_W2 + _D_OUT].set(Wr2.T)
    b2 = jnp.zeros((1, 2 * _W2), jnp.float32)
    b2 = b2.at[0, _W2:_W2 + _D_OUT].set(bl2)

    grid = (_N2 // _R,)
    q0p, rp, invc = pl.pallas_call(
        _dense_body,
        grid=grid,
        in_specs=[
            pl.BlockSpec((_R, _W1), lambda i: (i, 0)),
            pl.BlockSpec((_R, _W1), lambda i: (i, 0)),
            pl.BlockSpec((2, _R, _W2), lambda i: (0, i, 0)),
            pl.BlockSpec((_R, _D_IN), lambda i: (i, 0)),
            pl.BlockSpec((_W1, _D_H), lambda i: (0, 0)),
            pl.BlockSpec((_W1, _D_H), lambda i: (0, 0)),
            pl.BlockSpec((_D_IN, _D_H), lambda i: (0, 0)),
            pl.BlockSpec((1, _D_H), lambda i: (0, 0)),
            pl.BlockSpec((1, _D_H), lambda i: (0, 0)),
            pl.BlockSpec((1, _D_H), lambda i: (0, 0)),
            pl.BlockSpec((_D_H, 2 * _W2), lambda i: (0, 0)),
            pl.BlockSpec((1, 2 * _W2), lambda i: (0, 0)),
        ],
        out_specs=[
            pl.BlockSpec((_R, _W2), lambda i: (i, 0)),
            pl.BlockSpec((_R, _W2), lambda i: (i, 0)),
            pl.BlockSpec((_R, _W2), lambda i: (i, 0)),
        ],
        out_shape=[
            jax.ShapeDtypeStruct((_N2, _W2), jnp.float32),
            jax.ShapeDtypeStruct((_N2, _W2), jnp.float32),
            jax.ShapeDtypeStruct((_N2, _W2), jnp.float32),
        ],
    )(acc_lo, acc_hi, cnt2, x_pad, wlo, whi, wrst, b1, g2, be2, w2, b2)

    acc2 = _sc_pass2(q0p, src2d, dst2d)

    outp = pl.pallas_call(
        _combine_body,
        grid=grid,
        in_specs=[
            pl.BlockSpec((2, _R, _W2), lambda i: (0, i, 0)),
            pl.BlockSpec((_R, _W2), lambda i: (i, 0)),
            pl.BlockSpec((_R, _W2), lambda i: (i, 0)),
        ],
        out_specs=pl.BlockSpec((_R, _W2), lambda i: (i, 0)),
        out_shape=jax.ShapeDtypeStruct((_N2, _W2), jnp.float32),
    )(acc2, invc, rp)

    return outp[:_N, :_D_OUT]


# speedup vs baseline: 1.6080x; 1.6080x over previous
import functools

import jax
import jax.numpy as jnp
from jax import lax
from jax.experimental import pallas as pl
from jax.experimental.pallas import tpu as pltpu
from jax.experimental.pallas import tpu_sc as plsc

_N = 10000
_E = 320000
_D_IN = 128
_D_H = 256
_D_OUT = 2

_C1 = 64
_NS1 = 4
_R1 = 80
_CHUNK = 128
_K2 = 80
_E_PAD1 = 16 * _R1 * _NS1 * _C1
_E_PAD2 = 32 * _K2 * _CHUNK
_N2 = 10240
_W1 = 64
_W2 = 16
_RPS = _N2 // 16
_NSLOT = 8

_mesh = plsc.VectorSubcoreMesh(core_axis_name="c", subcore_axis_name="s")
_sc_params = pltpu.CompilerParams(use_tc_tiling_on_sc=False)


def _fill_rows(buf, value):
    n, w = buf.shape
    vec = jnp.full((16,), value, jnp.float32)

    def row(i, carry):
        for j in range(w // 16):
            buf[i, pl.ds(j * 16, 16)] = vec
        return carry

    lax.fori_loop(0, n, row, 0)


def _blast_stripe(buf, acc_sh, s):
    n = buf.shape[0]
    for t in range(_RPS // n):
        pltpu.sync_copy(buf, acc_sh.at[pl.ds(s * _RPS + t * n, n)])


def _edge_rounds(table, src_v, dst_v, bufs, acc_sh, sem_g, sem_s, nchunks,
                 cnt_cb=None, cnt_drain=None):
    nrounds = nchunks // _NSLOT

    for j in range(_NSLOT):
        pltpu.async_copy(table.at[src_v.at[j]], bufs[j], sem_g)

    def body(i, carry):
        k0 = i * _NSLOT
        for j in range(_NSLOT):
            k = k0 + j
            pltpu.make_async_copy(table.at[src_v.at[k]], bufs[j], sem_g).wait()
            pltpu.async_copy(bufs[j], acc_sh.at[dst_v.at[k]], sem_s, add=True)
            if cnt_cb is not None:
                cnt_cb(j, k)
        for j in range(_NSLOT):
            pltpu.make_async_copy(
                bufs[j], acc_sh.at[dst_v.at[k0 + j]], sem_s).wait()
        if cnt_drain is not None:
            cnt_drain(k0)

        @pl.when(i < nrounds - 1)
        def _():
            for j in range(_NSLOT):
                pltpu.async_copy(
                    table.at[src_v.at[k0 + _NSLOT + j]], bufs[j], sem_g)

        return carry

    lax.fori_loop(0, nrounds, body, 0)


@functools.partial(
    pl.kernel,
    mesh=_mesh,
    compiler_params=_sc_params,
    out_type=(
        jax.ShapeDtypeStruct((_N2, _W1), jnp.float32),
        jax.ShapeDtypeStruct((_N2, _W1), jnp.float32),
        jax.ShapeDtypeStruct((2, _N2, _W2), jnp.float32),
    ),
    scratch_types=[
        [pltpu.VMEM((2, _NS1, _C1), jnp.int32)] * 2,
        [pltpu.VMEM((_C1, _W1), jnp.float32)] * (2 * _NS1),
        pltpu.VMEM((_C1, _W2), jnp.float32),
        pltpu.VMEM((64, _W2), jnp.float32),
        pltpu.VMEM_SHARED((_N2, _W1), jnp.float32),
        pltpu.VMEM_SHARED((_N2, _W2), jnp.float32),
        pltpu.VMEM_SHARED((_N2, _W1), jnp.float32),
        [pltpu.SemaphoreType.DMA] * 2,
        [pltpu.SemaphoreType.DMA] * 2,
        pltpu.SemaphoreType.DMA,
        pltpu.SemaphoreType.DMA,
    ],
)
def _sc_pass1(t0_hbm, t1_hbm, idx_hbm, out_lo, out_hi, out_cnt,
              ibanks, bufs_flat, ones_v, zeros_v, acc_sh, cnt_sh, tbl_sh,
              sem_i, sem_g, sem_s, sem_c):
    c = lax.axis_index("c")
    s = lax.axis_index("s")
    bufs = [bufs_flat[:_NS1], bufs_flat[_NS1:]]

    def stage(table):
        for t in range(_RPS // _C1):
            rows = pl.ds(s * _RPS + t * _C1, _C1)
            pltpu.sync_copy(table.at[rows], bufs[1][0])
            pltpu.sync_copy(bufs[1][0], tbl_sh.at[rows])

    @pl.when(c == 0)
    def _():
        stage(t0_hbm)

    @pl.when(c == 1)
    def _():
        stage(t1_hbm)

    _fill_rows(bufs[0][0], 0.0)
    _fill_rows(ones_v, 1.0)
    _fill_rows(zeros_v, 0.0)
    _blast_stripe(bufs[0][0], acc_sh, s)
    _blast_stripe(zeros_v, cnt_sh, s)
    plsc.subcore_barrier()

    def run(table):
        pltpu.sync_copy(idx_hbm.at[s, 0], ibanks[0])
        for j in range(_NS1):
            pltpu.async_copy(
                table.at[ibanks[0].at[0, j]], bufs[0][j], sem_g[0])
        pltpu.async_copy(idx_hbm.at[s, 1], ibanks[1], sem_i[1])

        def half_round(r, x):
            y = 1 - x

            @pl.when(r + 1 < _R1)
            def _():
                pltpu.make_async_copy(
                    idx_hbm.at[s, 0], ibanks[y], sem_i[y]).wait()
                for j in range(_NS1):
                    pltpu.async_copy(
                        table.at[ibanks[y].at[0, j]], bufs[y][j], sem_g[y])

            for j in range(_NS1):
                pltpu.make_async_copy(
                    table.at[ibanks[x].at[0, j]], bufs[x][j], sem_g[x]).wait()
                pltpu.async_copy(
                    bufs[x][j], acc_sh.at[ibanks[x].at[1, j]], sem_s, add=True)

                @pl.when(c == j % 2)
                def _():
                    pltpu.async_copy(
                        ones_v, cnt_sh.at[ibanks[x].at[1, j]], sem_c, add=True)

            for j in range(_NS1):
                pltpu.make_async_copy(
                    bufs[x][j], acc_sh.at[ibanks[x].at[1, j]], sem_s).wait()
            for _unused in range(_NS1 // 2):
                pltpu.make_async_copy(
                    ones_v, cnt_sh.at[ibanks[x].at[1, 0]], sem_c).wait()

            @pl.when(r + 2 < _R1)
            def _():
                pltpu.async_copy(idx_hbm.at[s, r + 2], ibanks[x], sem_i[x])

        def body(i, carry):
            half_round(2 * i, 0)
            half_round(2 * i + 1, 1)
            return carry

        lax.fori_loop(0, _R1 // 2, body, 0)

    run(tbl_sh)

    plsc.subcore_barrier()

    rows = pl.ds(s * _RPS, _RPS)

    @pl.when(c == 0)
    def _():
        pltpu.sync_copy(acc_sh.at[rows], out_lo.at[rows])

    @pl.when(c == 1)
    def _():
        pltpu.sync_copy(acc_sh.at[rows], out_hi.at[rows])

    pltpu.sync_copy(cnt_sh.at[rows], out_cnt.at[c, rows])


@functools.partial(
    pl.kernel,
    mesh=_mesh,
    compiler_params=_sc_params,
    out_type=jax.ShapeDtypeStruct((2, _N2, _W2), jnp.float32),
    scratch_types=[
        pltpu.VMEM((_K2, _CHUNK), jnp.int32),
        pltpu.VMEM((_K2, _CHUNK), jnp.int32),
        [pltpu.VMEM((_CHUNK, _W2), jnp.float32)] * _NSLOT,
        pltpu.VMEM_SHARED((_N2, _W2), jnp.float32),
        pltpu.SemaphoreType.DMA,
        pltpu.SemaphoreType.DMA,
    ],
)
def _sc_pass2(q_hbm, src_hbm, dst_hbm, out_hbm,
              src_v, dst_v, bufs, acc_sh, sem_g, sem_s):
    c = lax.axis_index("c")
    s = lax.axis_index("s")
    wid = s * 2 + c

    pltpu.sync_copy(src_hbm.at[pl.ds(wid * _K2, _K2)], src_v)
    pltpu.sync_copy(dst_hbm.at[pl.ds(wid * _K2, _K2)], dst_v)
    _fill_rows(bufs[0], 0.0)
    _blast_stripe(bufs[0], acc_sh, s)
    plsc.subcore_barrier()

    _edge_rounds(q_hbm, src_v, dst_v, bufs, acc_sh, sem_g, sem_s, _K2)

    plsc.subcore_barrier()
    rows = pl.ds(s * _RPS, _RPS)
    pltpu.sync_copy(acc_sh.at[rows], out_hbm.at[c, rows])


_R = 640


def _dense_body(lo_ref, hi_ref, cnt_ref, x_ref, wlo_ref, whi_ref, wrst_ref,
                b1_ref, gamma_ref, beta_ref, w2_ref, b2_ref,
                q0_ref, r_ref, invc_ref):
    cnt = cnt_ref[0, :, 0:1] + cnt_ref[1, :, 0:1]
    inv = 1.0 / jnp.maximum(cnt, 1.0)
    x1 = (jnp.dot(lo_ref[...] * inv, wlo_ref[...],
                  preferred_element_type=jnp.float32)
          + jnp.dot(hi_ref[...] * inv, whi_ref[...],
                    preferred_element_type=jnp.float32)
          + jnp.dot(x_ref[...], wrst_ref[...],
                    preferred_element_type=jnp.float32)
          + b1_ref[...])
    mu = jnp.mean(x1, axis=-1, keepdims=True)
    var = jnp.mean((x1 - mu) ** 2, axis=-1, keepdims=True)
    xn = (x1 - mu) * lax.rsqrt(var + 1e-5) * gamma_ref[...] + beta_ref[...]
    h = jnp.where(xn > 0, xn, jnp.exp(xn) - 1.0)
    qr = jnp.dot(h, w2_ref[...], preferred_element_type=jnp.float32) \
        + b2_ref[...]
    q0_ref[...] = qr[:, :_W2]
    r_ref[...] = qr[:, _W2:]
    invc_ref[...] = jnp.broadcast_to(inv, (_R, _W2))


def _combine_body(acc2_ref, invc_ref, r_ref, out_ref):
    out_ref[...] = (acc2_ref[0] + acc2_ref[1]) * invc_ref[...] + r_ref[...]


def kernel(x, edge_index, Wl1, bl1, Wr1, Ws, bs, gamma, beta, Wl2, bl2, Wr2):
    src = edge_index[0]
    dst = edge_index[1]
    pad1 = _E_PAD1 - _E
    src_p1 = jnp.concatenate([src, jnp.zeros((pad1,), jnp.int32)])
    dst_p1 = jnp.concatenate([dst, jnp.full((pad1,), _N, jnp.int32)])
    pad2 = _E_PAD2 - _E
    src_p2 = jnp.concatenate([src, jnp.zeros((pad2,), jnp.int32)])
    dst_p2 = jnp.concatenate([dst, jnp.full((pad2,), _N, jnp.int32)])
    src2d = src_p2.reshape(32 * _K2, _CHUNK)
    dst2d = dst_p2.reshape(32 * _K2, _CHUNK)
    src4 = src_p1.reshape(16, _R1, _NS1, _C1)
    dst4 = dst_p1.reshape(16, _R1, _NS1, _C1)
    idx_blk = jnp.stack([src4, dst4], axis=2)

    t0 = jnp.zeros((_N2, _W1), jnp.float32).at[:_N].set(x[:, :64])
    t1 = jnp.zeros((_N2, _W1), jnp.float32).at[:_N].set(x[:, 64:128])
    x_pad = jnp.zeros((_N2, _D_IN), jnp.float32).at[:_N].set(x)

    acc_lo, acc_hi, cnt2 = _sc_pass1(t0, t1, idx_blk)

    wl1t = Wl1.T
    wlo = wl1t[:64]
    whi = wl1t[64:128]
    wrst = (Wr1 + Ws).T
    b1 = (bl1 + bs).reshape(1, _D_H)
    g2 = gamma.reshape(1, _D_H)
    be2 = beta.reshape(1, _D_H)
    w2 = jnp.zeros((_D_H, 2 * _W2), jnp.float32)
    w2 = w2.at[:, 0:_D_OUT].set(Wl2.T).at[:, _W2:_W2 + _D_OUT].set(Wr2.T)
    b2 = jnp.zeros((1, 2 * _W2), jnp.float32)
    b2 = b2.at[0, _W2:_W2 + _D_OUT].set(bl2)

    grid = (_N2 // _R,)
    q0p, rp, invc = pl.pallas_call(
        _dense_body,
        grid=grid,
        in_specs=[
            pl.BlockSpec((_R, _W1), lambda i: (i, 0)),
            pl.BlockSpec((_R, _W1), lambda i: (i, 0)),
            pl.BlockSpec((2, _R, _W2), lambda i: (0, i, 0)),
            pl.BlockSpec((_R, _D_IN), lambda i: (i, 0)),
            pl.BlockSpec((_W1, _D_H), lambda i: (0, 0)),
            pl.BlockSpec((_W1, _D_H), lambda i: (0, 0)),
            pl.BlockSpec((_D_IN, _D_H), lambda i: (0, 0)),
            pl.BlockSpec((1, _D_H), lambda i: (0, 0)),
            pl.BlockSpec((1, _D_H), lambda i: (0, 0)),
            pl.BlockSpec((1, _D_H), lambda i: (0, 0)),
            pl.BlockSpec((_D_H, 2 * _W2), lambda i: (0, 0)),
            pl.BlockSpec((1, 2 * _W2), lambda i: (0, 0)),
        ],
        out_specs=[
            pl.BlockSpec((_R, _W2), lambda i: (i, 0)),
            pl.BlockSpec((_R, _W2), lambda i: (i, 0)),
            pl.BlockSpec((_R, _W2), lambda i: (i, 0)),
        ],
        out_shape=[
            jax.ShapeDtypeStruct((_N2, _W2), jnp.float32),
            jax.ShapeDtypeStruct((_N2, _W2), jnp.float32),
            jax.ShapeDtypeStruct((_N2, _W2), jnp.float32),
        ],
    )(acc_lo, acc_hi, cnt2, x_pad, wlo, whi, wrst, b1, g2, be2, w2, b2)

    acc2 = _sc_pass2(q0p, src2d, dst2d)

    outp = pl.pallas_call(
        _combine_body,
        grid=grid,
        in_specs=[
            pl.BlockSpec((2, _R, _W2), lambda i: (0, i, 0)),
            pl.BlockSpec((_R, _W2), lambda i: (i, 0)),
            pl.BlockSpec((_R, _W2), lambda i: (i, 0)),
        ],
        out_specs=pl.BlockSpec((_R, _W2), lambda i: (i, 0)),
        out_shape=jax.ShapeDtypeStruct((_N2, _W2), jnp.float32),
    )(acc2, invc, rp)

    return outp[:_N, :_D_OUT]


# speedup vs baseline: 1.7525x; 1.0899x over previous
import functools

import jax
import jax.numpy as jnp
from jax import lax
from jax.experimental import pallas as pl
from jax.experimental.pallas import tpu as pltpu
from jax.experimental.pallas import tpu_sc as plsc

_N = 10000
_E = 320000
_D_IN = 128
_D_H = 256
_D_OUT = 2

_C1 = 64
_NS1 = 4
_R1 = 80
_CHUNK = 128
_K2 = 80
_E_PAD1 = 16 * _R1 * _NS1 * _C1
_E_PAD2 = 32 * _K2 * _CHUNK
_N2 = 10240
_W1 = 64
_W2 = 16
_RPS = _N2 // 16
_NSLOT = 8

_mesh = plsc.VectorSubcoreMesh(core_axis_name="c", subcore_axis_name="s")
_sc_params = pltpu.CompilerParams(use_tc_tiling_on_sc=False)


def _fill_rows(buf, value):
    n, w = buf.shape
    vec = jnp.full((16,), value, jnp.float32)

    def row(i, carry):
        for j in range(w // 16):
            buf[i, pl.ds(j * 16, 16)] = vec
        return carry

    lax.fori_loop(0, n, row, 0)


def _blast_stripe(buf, acc_sh, s):
    n = buf.shape[0]
    for t in range(_RPS // n):
        pltpu.sync_copy(buf, acc_sh.at[pl.ds(s * _RPS + t * n, n)])


def _edge_rounds(table, src_v, dst_v, bufs, acc_sh, sem_g, sem_s, nchunks,
                 cnt_cb=None, cnt_drain=None):
    nrounds = nchunks // _NSLOT

    for j in range(_NSLOT):
        pltpu.async_copy(table.at[src_v.at[j]], bufs[j], sem_g)

    def body(i, carry):
        k0 = i * _NSLOT
        for j in range(_NSLOT):
            k = k0 + j
            pltpu.make_async_copy(table.at[src_v.at[k]], bufs[j], sem_g).wait()
            pltpu.async_copy(bufs[j], acc_sh.at[dst_v.at[k]], sem_s, add=True)
            if cnt_cb is not None:
                cnt_cb(j, k)
        for j in range(_NSLOT):
            pltpu.make_async_copy(
                bufs[j], acc_sh.at[dst_v.at[k0 + j]], sem_s).wait()
        if cnt_drain is not None:
            cnt_drain(k0)

        @pl.when(i < nrounds - 1)
        def _():
            for j in range(_NSLOT):
                pltpu.async_copy(
                    table.at[src_v.at[k0 + _NSLOT + j]], bufs[j], sem_g)

        return carry

    lax.fori_loop(0, nrounds, body, 0)


@functools.partial(
    pl.kernel,
    mesh=_mesh,
    compiler_params=_sc_params,
    out_type=(
        jax.ShapeDtypeStruct((_N2, _W1), jnp.float32),
        jax.ShapeDtypeStruct((_N2, _W1), jnp.float32),
        jax.ShapeDtypeStruct((2, _N2, _W2), jnp.float32),
    ),
    scratch_types=[
        [pltpu.VMEM((2, _NS1, _C1), jnp.int32)] * 2,
        [pltpu.VMEM((_C1, _W1), jnp.float32)] * (2 * _NS1),
        pltpu.VMEM((_C1, _W2), jnp.float32),
        pltpu.VMEM((64, _W2), jnp.float32),
        pltpu.VMEM_SHARED((_N2, _W1), jnp.float32),
        pltpu.VMEM_SHARED((_N2, _W2), jnp.float32),
        pltpu.VMEM_SHARED((_N2, _W1), jnp.float32),
        [pltpu.SemaphoreType.DMA] * 2,
        [pltpu.SemaphoreType.DMA] * 2,
        pltpu.SemaphoreType.DMA,
        pltpu.SemaphoreType.DMA,
    ],
)
def _sc_pass1(t0_hbm, t1_hbm, idx_hbm, out_lo, out_hi, out_cnt,
              ibanks, bufs_flat, ones_v, zeros_v, acc_sh, cnt_sh, tbl_sh,
              sem_i, sem_g, sem_s, sem_c):
    c = lax.axis_index("c")
    s = lax.axis_index("s")
    bufs = [bufs_flat[:_NS1], bufs_flat[_NS1:]]

    def stage(table):
        for t in range(_RPS // _C1):
            rows = pl.ds(s * _RPS + t * _C1, _C1)
            pltpu.sync_copy(table.at[rows], bufs[1][0])
            pltpu.sync_copy(bufs[1][0], tbl_sh.at[rows])

    @pl.when(c == 0)
    def _():
        stage(t0_hbm)

    @pl.when(c == 1)
    def _():
        stage(t1_hbm)

    _fill_rows(bufs[0][0], 0.0)
    _fill_rows(ones_v, 1.0)
    _fill_rows(zeros_v, 0.0)
    _blast_stripe(bufs[0][0], acc_sh, s)
    _blast_stripe(zeros_v, cnt_sh, s)
    plsc.subcore_barrier()

    def run(table):
        pltpu.sync_copy(idx_hbm.at[s, 0], ibanks[0])
        for j in range(_NS1):
            pltpu.async_copy(
                table.at[ibanks[0].at[0, j]], bufs[0][j], sem_g[0])
        pltpu.async_copy(idx_hbm.at[s, 1], ibanks[1], sem_i[1])

        def half_round(r, x):
            y = 1 - x

            @pl.when(r + 1 < _R1)
            def _():
                pltpu.make_async_copy(
                    idx_hbm.at[s, 0], ibanks[y], sem_i[y]).wait()
                for j in range(_NS1):
                    pltpu.async_copy(
                        table.at[ibanks[y].at[0, j]], bufs[y][j], sem_g[y])

            for j in range(_NS1):
                pltpu.make_async_copy(
                    table.at[ibanks[x].at[0, j]], bufs[x][j], sem_g[x]).wait()
                pltpu.async_copy(
                    bufs[x][j], acc_sh.at[ibanks[x].at[1, j]], sem_s, add=True)

                @pl.when(c == j % 2)
                def _():
                    pltpu.async_copy(
                        ones_v, cnt_sh.at[ibanks[x].at[1, j]], sem_c, add=True)

            for j in range(_NS1):
                pltpu.make_async_copy(
                    bufs[x][j], acc_sh.at[ibanks[x].at[1, j]], sem_s).wait()
            for _unused in range(_NS1 // 2):
                pltpu.make_async_copy(
                    ones_v, cnt_sh.at[ibanks[x].at[1, 0]], sem_c).wait()

            @pl.when(r + 2 < _R1)
            def _():
                pltpu.async_copy(idx_hbm.at[s, r + 2], ibanks[x], sem_i[x])

        def body(i, carry):
            half_round(2 * i, 0)
            half_round(2 * i + 1, 1)
            return carry

        lax.fori_loop(0, _R1 // 2, body, 0)

    run(tbl_sh)

    plsc.subcore_barrier()

    rows = pl.ds(s * _RPS, _RPS)

    @pl.when(c == 0)
    def _():
        pltpu.sync_copy(acc_sh.at[rows], out_lo.at[rows])

    @pl.when(c == 1)
    def _():
        pltpu.sync_copy(acc_sh.at[rows], out_hi.at[rows])

    pltpu.sync_copy(cnt_sh.at[rows], out_cnt.at[c, rows])


@functools.partial(
    pl.kernel,
    mesh=_mesh,
    compiler_params=_sc_params,
    out_type=jax.ShapeDtypeStruct((2, _N2, _W2), jnp.float32),
    scratch_types=[
        pltpu.VMEM((_K2, _CHUNK), jnp.int32),
        pltpu.VMEM((_K2, _CHUNK), jnp.int32),
        [pltpu.VMEM((_CHUNK, _W2), jnp.float32)] * _NSLOT,
        pltpu.VMEM_SHARED((_N2, _W2), jnp.float32),
        pltpu.VMEM_SHARED((_N2, _W2), jnp.float32),
        pltpu.SemaphoreType.DMA,
        pltpu.SemaphoreType.DMA,
    ],
)
def _sc_pass2(q_hbm, src_hbm, dst_hbm, out_hbm,
              src_v, dst_v, bufs, acc_sh, tbl_sh, sem_g, sem_s):
    c = lax.axis_index("c")
    s = lax.axis_index("s")
    wid = s * 2 + c

    pltpu.sync_copy(src_hbm.at[pl.ds(wid * _K2, _K2)], src_v)
    pltpu.sync_copy(dst_hbm.at[pl.ds(wid * _K2, _K2)], dst_v)
    for t in range(_RPS // _CHUNK):
        rows = pl.ds(s * _RPS + t * _CHUNK, _CHUNK)
        pltpu.sync_copy(q_hbm.at[rows], bufs[1])
        pltpu.sync_copy(bufs[1], tbl_sh.at[rows])
    _fill_rows(bufs[0], 0.0)
    _blast_stripe(bufs[0], acc_sh, s)
    plsc.subcore_barrier()

    _edge_rounds(tbl_sh, src_v, dst_v, bufs, acc_sh, sem_g, sem_s, _K2)

    plsc.subcore_barrier()
    rows = pl.ds(s * _RPS, _RPS)
    pltpu.sync_copy(acc_sh.at[rows], out_hbm.at[c, rows])


_R = 640


def _dense_body(lo_ref, hi_ref, cnt_ref, x_ref, wlo_ref, whi_ref, wrst_ref,
                b1_ref, gamma_ref, beta_ref, w2_ref, b2_ref,
                q0_ref, r_ref, invc_ref):
    cnt = cnt_ref[0, :, 0:1] + cnt_ref[1, :, 0:1]
    inv = 1.0 / jnp.maximum(cnt, 1.0)
    x1 = (jnp.dot(lo_ref[...] * inv, wlo_ref[...],
                  preferred_element_type=jnp.float32)
          + jnp.dot(hi_ref[...] * inv, whi_ref[...],
                    preferred_element_type=jnp.float32)
          + jnp.dot(x_ref[...], wrst_ref[...],
                    preferred_element_type=jnp.float32)
          + b1_ref[...])
    mu = jnp.mean(x1, axis=-1, keepdims=True)
    var = jnp.mean((x1 - mu) ** 2, axis=-1, keepdims=True)
    xn = (x1 - mu) * lax.rsqrt(var + 1e-5) * gamma_ref[...] + beta_ref[...]
    h = jnp.where(xn > 0, xn, jnp.exp(xn) - 1.0)
    qr = jnp.dot(h, w2_ref[...], preferred_element_type=jnp.float32) \
        + b2_ref[...]
    q0_ref[...] = qr[:, :_W2]
    r_ref[...] = qr[:, _W2:]
    invc_ref[...] = jnp.broadcast_to(inv, (_R, _W2))


def _combine_body(acc2_ref, invc_ref, r_ref, out_ref):
    out_ref[...] = (acc2_ref[0] + acc2_ref[1]) * invc_ref[...] + r_ref[...]


def kernel(x, edge_index, Wl1, bl1, Wr1, Ws, bs, gamma, beta, Wl2, bl2, Wr2):
    src = edge_index[0]
    dst = edge_index[1]
    pad1 = _E_PAD1 - _E
    src_p1 = jnp.concatenate([src, jnp.zeros((pad1,), jnp.int32)])
    dst_p1 = jnp.concatenate([dst, jnp.full((pad1,), _N, jnp.int32)])
    pad2 = _E_PAD2 - _E
    src_p2 = jnp.concatenate([src, jnp.zeros((pad2,), jnp.int32)])
    dst_p2 = jnp.concatenate([dst, jnp.full((pad2,), _N, jnp.int32)])
    src2d = src_p2.reshape(32 * _K2, _CHUNK)
    dst2d = dst_p2.reshape(32 * _K2, _CHUNK)
    src4 = src_p1.reshape(16, _R1, _NS1, _C1)
    dst4 = dst_p1.reshape(16, _R1, _NS1, _C1)
    idx_blk = jnp.stack([src4, dst4], axis=2)

    t0 = jnp.zeros((_N2, _W1), jnp.float32).at[:_N].set(x[:, :64])
    t1 = jnp.zeros((_N2, _W1), jnp.float32).at[:_N].set(x[:, 64:128])
    x_pad = jnp.zeros((_N2, _D_IN), jnp.float32).at[:_N].set(x)

    acc_lo, acc_hi, cnt2 = _sc_pass1(t0, t1, idx_blk)

    wl1t = Wl1.T
    wlo = wl1t[:64]
    whi = wl1t[64:128]
    wrst = (Wr1 + Ws).T
    b1 = (bl1 + bs).reshape(1, _D_H)
    g2 = gamma.reshape(1, _D_H)
    be2 = beta.reshape(1, _D_H)
    w2 = jnp.zeros((_D_H, 2 * _W2), jnp.float32)
    w2 = w2.at[:, 0:_D_OUT].set(Wl2.T).at[:, _W2:_W2 + _D_OUT].set(Wr2.T)
    b2 = jnp.zeros((1, 2 * _W2), jnp.float32)
    b2 = b2.at[0, _W2:_W2 + _D_OUT].set(bl2)

    grid = (_N2 // _R,)
    q0p, rp, invc = pl.pallas_call(
        _dense_body,
        grid=grid,
        in_specs=[
            pl.BlockSpec((_R, _W1), lambda i: (i, 0)),
            pl.BlockSpec((_R, _W1), lambda i: (i, 0)),
            pl.BlockSpec((2, _R, _W2), lambda i: (0, i, 0)),
            pl.BlockSpec((_R, _D_IN), lambda i: (i, 0)),
            pl.BlockSpec((_W1, _D_H), lambda i: (0, 0)),
            pl.BlockSpec((_W1, _D_H), lambda i: (0, 0)),
            pl.BlockSpec((_D_IN, _D_H), lambda i: (0, 0)),
            pl.BlockSpec((1, _D_H), lambda i: (0, 0)),
            pl.BlockSpec((1, _D_H), lambda i: (0, 0)),
            pl.BlockSpec((1, _D_H), lambda i: (0, 0)),
            pl.BlockSpec((_D_H, 2 * _W2), lambda i: (0, 0)),
            pl.BlockSpec((1, 2 * _W2), lambda i: (0, 0)),
        ],
        out_specs=[
            pl.BlockSpec((_R, _W2), lambda i: (i, 0)),
            pl.BlockSpec((_R, _W2), lambda i: (i, 0)),
            pl.BlockSpec((_R, _W2), lambda i: (i, 0)),
        ],
        out_shape=[
            jax.ShapeDtypeStruct((_N2, _W2), jnp.float32),
            jax.ShapeDtypeStruct((_N2, _W2), jnp.float32),
            jax.ShapeDtypeStruct((_N2, _W2), jnp.float32),
        ],
    )(acc_lo, acc_hi, cnt2, x_pad, wlo, whi, wrst, b1, g2, be2, w2, b2)

    acc2 = _sc_pass2(q0p, src2d, dst2d)

    outp = pl.pallas_call(
        _combine_body,
        grid=grid,
        in_specs=[
            pl.BlockSpec((2, _R, _W2), lambda i: (0, i, 0)),
            pl.BlockSpec((_R, _W2), lambda i: (i, 0)),
            pl.BlockSpec((_R, _W2), lambda i: (i, 0)),
        ],
        out_specs=pl.BlockSpec((_R, _W2), lambda i: (i, 0)),
        out_shape=jax.ShapeDtypeStruct((_N2, _W2), jnp.float32),
    )(acc2, invc, rp)

    return outp[:_N, :_D_OUT]


# speedup vs baseline: 1.8165x; 1.0365x over previous
import functools

import jax
import jax.numpy as jnp
from jax import lax
from jax.experimental import pallas as pl
from jax.experimental.pallas import tpu as pltpu
from jax.experimental.pallas import tpu_sc as plsc

_N = 10000
_E = 320000
_D_IN = 128
_D_H = 256
_D_OUT = 2

_C1 = 64
_NS1 = 4
_R1 = 80
_CHUNK = 128
_K2 = 80
_E_PAD1 = 16 * _R1 * _NS1 * _C1
_E_PAD2 = 32 * _K2 * _CHUNK
_N2 = 10240
_W1 = 64
_W2 = 16
_RPS = _N2 // 16
_NSLOT = 8

_mesh = plsc.VectorSubcoreMesh(core_axis_name="c", subcore_axis_name="s")
_sc_params = pltpu.CompilerParams(use_tc_tiling_on_sc=False)


def _fill_rows(buf, value):
    n, w = buf.shape
    vec = jnp.full((16,), value, jnp.float32)

    def row(i, carry):
        for j in range(w // 16):
            buf[i, pl.ds(j * 16, 16)] = vec
        return carry

    lax.fori_loop(0, n, row, 0)


def _blast_stripe(buf, acc_sh, s):
    n = buf.shape[0]
    for t in range(_RPS // n):
        pltpu.sync_copy(buf, acc_sh.at[pl.ds(s * _RPS + t * n, n)])


def _edge_rounds(table, src_v, dst_v, bufs, acc_sh, sem_g, sem_s, nchunks,
                 cnt_cb=None, cnt_drain=None):
    nrounds = nchunks // _NSLOT

    for j in range(_NSLOT):
        pltpu.async_copy(table.at[src_v.at[j]], bufs[j], sem_g)

    def body(i, carry):
        k0 = i * _NSLOT
        for j in range(_NSLOT):
            k = k0 + j
            pltpu.make_async_copy(table.at[src_v.at[k]], bufs[j], sem_g).wait()
            pltpu.async_copy(bufs[j], acc_sh.at[dst_v.at[k]], sem_s, add=True)
            if cnt_cb is not None:
                cnt_cb(j, k)
        for j in range(_NSLOT):
            pltpu.make_async_copy(
                bufs[j], acc_sh.at[dst_v.at[k0 + j]], sem_s).wait()
        if cnt_drain is not None:
            cnt_drain(k0)

        @pl.when(i < nrounds - 1)
        def _():
            for j in range(_NSLOT):
                pltpu.async_copy(
                    table.at[src_v.at[k0 + _NSLOT + j]], bufs[j], sem_g)

        return carry

    lax.fori_loop(0, nrounds, body, 0)


@functools.partial(
    pl.kernel,
    mesh=_mesh,
    compiler_params=_sc_params,
    out_type=(
        jax.ShapeDtypeStruct((_N2, _W1), jnp.float32),
        jax.ShapeDtypeStruct((_N2, _W1), jnp.float32),
        jax.ShapeDtypeStruct((2, _N2, _W2), jnp.float32),
    ),
    scratch_types=[
        [pltpu.VMEM((2, _NS1, _C1), jnp.int32)] * 2,
        [pltpu.VMEM((_C1, _W1), jnp.float32)] * (2 * _NS1),
        pltpu.VMEM((_C1, _W2), jnp.float32),
        pltpu.VMEM((64, _W2), jnp.float32),
        pltpu.VMEM_SHARED((_N2, _W1), jnp.float32),
        pltpu.VMEM_SHARED((_N2, _W2), jnp.float32),
        pltpu.VMEM_SHARED((_N2, _W1), jnp.float32),
        [pltpu.SemaphoreType.DMA] * 2,
        [pltpu.SemaphoreType.DMA] * 2,
        pltpu.SemaphoreType.DMA,
        pltpu.SemaphoreType.DMA,
    ],
)
def _sc_pass1(x_hbm, idx_hbm, out_lo, out_hi, out_cnt,
              ibanks, bufs_flat, ones_v, zeros_v, acc_sh, cnt_sh, tbl_sh,
              sem_i, sem_g, sem_s, sem_c):
    c = lax.axis_index("c")
    s = lax.axis_index("s")
    bufs = [bufs_flat[:_NS1], bufs_flat[_NS1:]]

    cols = pl.ds(c * _W1, _W1)

    def stage_chunk(r0, buf):
        pltpu.sync_copy(x_hbm.at[pl.ds(r0, buf.shape[0]), cols], buf)
        pltpu.sync_copy(buf, tbl_sh.at[pl.ds(r0, buf.shape[0])])

    @pl.when(s < 15)
    def _():
        for t in range(_RPS // _C1):
            stage_chunk(s * _RPS + t * _C1, bufs[1][0])

    @pl.when(s == 15)
    def _():
        for t in range((_N - 15 * _RPS) // _C1):
            stage_chunk(15 * _RPS + t * _C1, bufs[1][0])
        stage_chunk(_N - 16, bufs[1][0].at[pl.ds(0, 16)])

    _fill_rows(bufs[0][0], 0.0)
    _fill_rows(ones_v, 1.0)
    _fill_rows(zeros_v, 0.0)
    _blast_stripe(bufs[0][0], acc_sh, s)
    _blast_stripe(zeros_v, cnt_sh, s)
    plsc.subcore_barrier()

    def run(table):
        pltpu.sync_copy(idx_hbm.at[s, 0], ibanks[0])
        for j in range(_NS1):
            pltpu.async_copy(
                table.at[ibanks[0].at[0, j]], bufs[0][j], sem_g[0])
        pltpu.async_copy(idx_hbm.at[s, 1], ibanks[1], sem_i[1])

        def half_round(r, x):
            y = 1 - x

            @pl.when(r + 1 < _R1)
            def _():
                pltpu.make_async_copy(
                    idx_hbm.at[s, 0], ibanks[y], sem_i[y]).wait()
                for j in range(_NS1):
                    pltpu.async_copy(
                        table.at[ibanks[y].at[0, j]], bufs[y][j], sem_g[y])

            for j in range(_NS1):
                pltpu.make_async_copy(
                    table.at[ibanks[x].at[0, j]], bufs[x][j], sem_g[x]).wait()
                pltpu.async_copy(
                    bufs[x][j], acc_sh.at[ibanks[x].at[1, j]], sem_s, add=True)

                @pl.when(c == j % 2)
                def _():
                    pltpu.async_copy(
                        ones_v, cnt_sh.at[ibanks[x].at[1, j]], sem_c, add=True)

            for j in range(_NS1):
                pltpu.make_async_copy(
                    bufs[x][j], acc_sh.at[ibanks[x].at[1, j]], sem_s).wait()
            for _unused in range(_NS1 // 2):
                pltpu.make_async_copy(
                    ones_v, cnt_sh.at[ibanks[x].at[1, 0]], sem_c).wait()

            @pl.when(r + 2 < _R1)
            def _():
                pltpu.async_copy(idx_hbm.at[s, r + 2], ibanks[x], sem_i[x])

        def body(i, carry):
            half_round(2 * i, 0)
            half_round(2 * i + 1, 1)
            return carry

        lax.fori_loop(0, _R1 // 2, body, 0)

    run(tbl_sh)

    plsc.subcore_barrier()

    rows = pl.ds(s * _RPS, _RPS)

    @pl.when(c == 0)
    def _():
        pltpu.sync_copy(acc_sh.at[rows], out_lo.at[rows])

    @pl.when(c == 1)
    def _():
        pltpu.sync_copy(acc_sh.at[rows], out_hi.at[rows])

    pltpu.sync_copy(cnt_sh.at[rows], out_cnt.at[c, rows])


@functools.partial(
    pl.kernel,
    mesh=_mesh,
    compiler_params=_sc_params,
    out_type=jax.ShapeDtypeStruct((2, _N2, _W2), jnp.float32),
    scratch_types=[
        pltpu.VMEM((_K2, _CHUNK), jnp.int32),
        pltpu.VMEM((_K2, _CHUNK), jnp.int32),
        [pltpu.VMEM((_CHUNK, _W2), jnp.float32)] * _NSLOT,
        pltpu.VMEM_SHARED((_N2, _W2), jnp.float32),
        pltpu.VMEM_SHARED((_N2, _W2), jnp.float32),
        pltpu.SemaphoreType.DMA,
        pltpu.SemaphoreType.DMA,
    ],
)
def _sc_pass2(q_hbm, src_hbm, dst_hbm, out_hbm,
              src_v, dst_v, bufs, acc_sh, tbl_sh, sem_g, sem_s):
    c = lax.axis_index("c")
    s = lax.axis_index("s")
    wid = s * 2 + c

    pltpu.sync_copy(src_hbm.at[pl.ds(wid * _K2, _K2)], src_v)
    pltpu.sync_copy(dst_hbm.at[pl.ds(wid * _K2, _K2)], dst_v)
    def stage_chunk(r0, buf):
        pltpu.sync_copy(q_hbm.at[pl.ds(r0, buf.shape[0])], buf)
        pltpu.sync_copy(buf, tbl_sh.at[pl.ds(r0, buf.shape[0])])

    @pl.when(s < 15)
    def _():
        for t in range(_RPS // _CHUNK):
            stage_chunk(s * _RPS + t * _CHUNK, bufs[1])

    @pl.when(s == 15)
    def _():
        for t in range((_N - 15 * _RPS) // _CHUNK):
            stage_chunk(15 * _RPS + t * _CHUNK, bufs[1])
        stage_chunk(_N - 16, bufs[1].at[pl.ds(0, 16)])
    _fill_rows(bufs[0], 0.0)
    _blast_stripe(bufs[0], acc_sh, s)
    plsc.subcore_barrier()

    _edge_rounds(tbl_sh, src_v, dst_v, bufs, acc_sh, sem_g, sem_s, _K2)

    plsc.subcore_barrier()
    rows = pl.ds(s * _RPS, _RPS)
    pltpu.sync_copy(acc_sh.at[rows], out_hbm.at[c, rows])


_R = 400


def _dense_body(lo_ref, hi_ref, cnt_ref, x_ref, wlo_ref, whi_ref, wrst_ref,
                b1_ref, gamma_ref, beta_ref, w2_ref, b2_ref,
                q0_ref, r_ref):
    cnt = cnt_ref[0, :, 0:1] + cnt_ref[1, :, 0:1]
    inv = 1.0 / jnp.maximum(cnt, 1.0)
    x1 = (jnp.dot(lo_ref[...] * inv, wlo_ref[...],
                  preferred_element_type=jnp.float32)
          + jnp.dot(hi_ref[...] * inv, whi_ref[...],
                    preferred_element_type=jnp.float32)
          + jnp.dot(x_ref[...], wrst_ref[...],
                    preferred_element_type=jnp.float32)
          + b1_ref[...])
    mu = jnp.mean(x1, axis=-1, keepdims=True)
    var = jnp.mean((x1 - mu) ** 2, axis=-1, keepdims=True)
    xn = (x1 - mu) * lax.rsqrt(var + 1e-5) * gamma_ref[...] + beta_ref[...]
    h = jnp.where(xn > 0, xn, jnp.exp(xn) - 1.0)
    qr = jnp.dot(h, w2_ref[...], preferred_element_type=jnp.float32) \
        + b2_ref[...]
    q0_ref[...] = qr[:, :_W2]
    r_ref[...] = qr[:, _W2:]


def _combine_body(acc2_ref, cnt_ref, r_ref, out_ref):
    cnt = cnt_ref[0, :, 0:1] + cnt_ref[1, :, 0:1]
    inv = 1.0 / jnp.maximum(cnt, 1.0)
    res = (acc2_ref[0] + acc2_ref[1]) * inv + r_ref[...]
    out_ref[...] = res[:, :_D_OUT]


def kernel(x, edge_index, Wl1, bl1, Wr1, Ws, bs, gamma, beta, Wl2, bl2, Wr2):
    src = edge_index[0]
    dst = edge_index[1]
    pad1 = _E_PAD1 - _E
    src_p = jnp.concatenate([src, jnp.zeros((pad1,), jnp.int32)])
    dst_p = jnp.concatenate([dst, jnp.full((pad1,), _N, jnp.int32)])
    src2d = src_p.reshape(32 * _K2, _CHUNK)
    dst2d = dst_p.reshape(32 * _K2, _CHUNK)
    src4 = src_p.reshape(16, _R1, _NS1, _C1)
    dst4 = dst_p.reshape(16, _R1, _NS1, _C1)
    idx_blk = jnp.stack([src4, dst4], axis=2)

    acc_lo, acc_hi, cnt2 = _sc_pass1(x, idx_blk)

    wl1t = Wl1.T
    wlo = wl1t[:64]
    whi = wl1t[64:128]
    wrst = (Wr1 + Ws).T
    b1 = (bl1 + bs).reshape(1, _D_H)
    g2 = gamma.reshape(1, _D_H)
    be2 = beta.reshape(1, _D_H)
    w2 = jnp.zeros((_D_H, 2 * _W2), jnp.float32)
    w2 = w2.at[:, 0:_D_OUT].set(Wl2.T).at[:, _W2:_W2 + _D_OUT].set(Wr2.T)
    b2 = jnp.zeros((1, 2 * _W2), jnp.float32)
    b2 = b2.at[0, _W2:_W2 + _D_OUT].set(bl2)

    grid = (_N // _R,)
    q0p, rp = pl.pallas_call(
        _dense_body,
        grid=grid,
        in_specs=[
            pl.BlockSpec((_R, _W1), lambda i: (i, 0)),
            pl.BlockSpec((_R, _W1), lambda i: (i, 0)),
            pl.BlockSpec((2, _R, _W2), lambda i: (0, i, 0)),
            pl.BlockSpec((_R, _D_IN), lambda i: (i, 0)),
            pl.BlockSpec((_W1, _D_H), lambda i: (0, 0)),
            pl.BlockSpec((_W1, _D_H), lambda i: (0, 0)),
            pl.BlockSpec((_D_IN, _D_H), lambda i: (0, 0)),
            pl.BlockSpec((1, _D_H), lambda i: (0, 0)),
            pl.BlockSpec((1, _D_H), lambda i: (0, 0)),
            pl.BlockSpec((1, _D_H), lambda i: (0, 0)),
            pl.BlockSpec((_D_H, 2 * _W2), lambda i: (0, 0)),
            pl.BlockSpec((1, 2 * _W2), lambda i: (0, 0)),
        ],
        out_specs=[
            pl.BlockSpec((_R, _W2), lambda i: (i, 0)),
            pl.BlockSpec((_R, _W2), lambda i: (i, 0)),
        ],
        out_shape=[
            jax.ShapeDtypeStruct((_N, _W2), jnp.float32),
            jax.ShapeDtypeStruct((_N, _W2), jnp.float32),
        ],
    )(acc_lo, acc_hi, cnt2, x, wlo, whi, wrst, b1, g2, be2, w2, b2)

    acc2 = _sc_pass2(q0p, src2d, dst2d)

    outp = pl.pallas_call(
        _combine_body,
        grid=grid,
        in_specs=[
            pl.BlockSpec((2, _R, _W2), lambda i: (0, i, 0)),
            pl.BlockSpec((2, _R, _W2), lambda i: (0, i, 0)),
            pl.BlockSpec((_R, _W2), lambda i: (i, 0)),
        ],
        out_specs=pl.BlockSpec((_R, _D_OUT), lambda i: (i, 0)),
        out_shape=jax.ShapeDtypeStruct((_N, _D_OUT), jnp.float32),
    )(acc2, cnt2, rp)

    return outp


# speedup vs baseline: 1.8991x; 1.0455x over previous
import functools

import jax
import jax.numpy as jnp
from jax import lax
from jax.experimental import pallas as pl
from jax.experimental.pallas import tpu as pltpu
from jax.experimental.pallas import tpu_sc as plsc

_N = 10000
_E = 320000
_D_IN = 128
_D_H = 256
_D_OUT = 2

_C1 = 64
_NS1 = 4
_R1 = 80
_CHUNK = 128
_K2 = 80
_E_PAD1 = 16 * _R1 * _NS1 * _C1
_E_PAD2 = 32 * _K2 * _CHUNK
_N2 = 10240
_W1 = 64
_W2 = 16
_RPS = _N2 // 16
_NSLOT = 8

_mesh = plsc.VectorSubcoreMesh(core_axis_name="c", subcore_axis_name="s")
_sc_params = pltpu.CompilerParams(use_tc_tiling_on_sc=False)


def _fill_rows(buf, value):
    n, w = buf.shape
    vec = jnp.full((16,), value, jnp.float32)

    def row(i, carry):
        for j in range(w // 16):
            buf[i, pl.ds(j * 16, 16)] = vec
        return carry

    lax.fori_loop(0, n, row, 0)


def _blast_stripe(buf, acc_sh, s):
    n = buf.shape[0]
    for t in range(_RPS // n):
        pltpu.sync_copy(buf, acc_sh.at[pl.ds(s * _RPS + t * n, n)])


def _edge_rounds(table, src_v, dst_v, bufs, acc_sh, sem_g, sem_s, nchunks,
                 cnt_cb=None, cnt_drain=None):
    nrounds = nchunks // _NSLOT

    for j in range(_NSLOT):
        pltpu.async_copy(table.at[src_v.at[j]], bufs[j], sem_g)

    def body(i, carry):
        k0 = i * _NSLOT
        for j in range(_NSLOT):
            k = k0 + j
            pltpu.make_async_copy(table.at[src_v.at[k]], bufs[j], sem_g).wait()
            pltpu.async_copy(bufs[j], acc_sh.at[dst_v.at[k]], sem_s, add=True)
            if cnt_cb is not None:
                cnt_cb(j, k)
        for j in range(_NSLOT):
            pltpu.make_async_copy(
                bufs[j], acc_sh.at[dst_v.at[k0 + j]], sem_s).wait()
        if cnt_drain is not None:
            cnt_drain(k0)

        @pl.when(i < nrounds - 1)
        def _():
            for j in range(_NSLOT):
                pltpu.async_copy(
                    table.at[src_v.at[k0 + _NSLOT + j]], bufs[j], sem_g)

        return carry

    lax.fori_loop(0, nrounds, body, 0)


@functools.partial(
    pl.kernel,
    mesh=_mesh,
    compiler_params=_sc_params,
    out_type=(
        jax.ShapeDtypeStruct((_N2, _W1), jnp.float32),
        jax.ShapeDtypeStruct((_N2, _W1), jnp.float32),
        jax.ShapeDtypeStruct((2, _N2, _W2), jnp.float32),
    ),
    scratch_types=[
        [pltpu.VMEM((_NS1, _C1), jnp.int32)] * 4,
        [pltpu.VMEM((_C1, _W1), jnp.float32)] * (2 * _NS1),
        pltpu.VMEM((_C1, _W2), jnp.float32),
        pltpu.VMEM((64, _W2), jnp.float32),
        pltpu.VMEM_SHARED((_N2, _W1), jnp.float32),
        pltpu.VMEM_SHARED((_N2, _W2), jnp.float32),
        pltpu.VMEM_SHARED((_N2, _W1), jnp.float32),
        [pltpu.SemaphoreType.DMA] * 2,
        [pltpu.SemaphoreType.DMA] * 2,
        pltpu.SemaphoreType.DMA,
        pltpu.SemaphoreType.DMA,
    ],
)
def _sc_pass1(x_hbm, sidx_hbm, didx_hbm, out_lo, out_hi, out_cnt,
              ibanks_flat, bufs_flat, ones_v, zeros_v, acc_sh, cnt_sh, tbl_sh,
              sem_i, sem_g, sem_s, sem_c):
    c = lax.axis_index("c")
    s = lax.axis_index("s")
    bufs = [bufs_flat[:_NS1], bufs_flat[_NS1:]]
    sbank = ibanks_flat[:2]
    dbank = ibanks_flat[2:]

    cols = pl.ds(c * _W1, _W1)

    def stage_chunk(r0, buf):
        pltpu.sync_copy(x_hbm.at[pl.ds(r0, buf.shape[0]), cols], buf)
        pltpu.sync_copy(buf, tbl_sh.at[pl.ds(r0, buf.shape[0])])

    @pl.when(s < 15)
    def _():
        for t in range(_RPS // _C1):
            stage_chunk(s * _RPS + t * _C1, bufs[1][0])

    @pl.when(s == 15)
    def _():
        for t in range((_N - 15 * _RPS) // _C1):
            stage_chunk(15 * _RPS + t * _C1, bufs[1][0])
        stage_chunk(_N - 16, bufs[1][0].at[pl.ds(0, 16)])

    _fill_rows(bufs[0][0], 0.0)
    _fill_rows(ones_v, 1.0)
    _fill_rows(zeros_v, 0.0)
    _blast_stripe(bufs[0][0], acc_sh, s)
    _blast_stripe(zeros_v, cnt_sh, s)
    plsc.subcore_barrier()

    def run(table):
        pltpu.sync_copy(sidx_hbm.at[s, 0], sbank[0])
        pltpu.sync_copy(didx_hbm.at[s, 0], dbank[0])
        for j in range(_NS1):
            pltpu.async_copy(table.at[sbank[0].at[j]], bufs[0][j], sem_g[0])
        pltpu.async_copy(sidx_hbm.at[s, 1], sbank[1], sem_i[1])
        pltpu.async_copy(didx_hbm.at[s, 1], dbank[1], sem_i[1])

        def half_round(r, x):
            y = 1 - x

            @pl.when(r + 1 < _R1)
            def _():
                pltpu.make_async_copy(
                    sidx_hbm.at[s, 0], sbank[y], sem_i[y]).wait()
                pltpu.make_async_copy(
                    didx_hbm.at[s, 0], dbank[y], sem_i[y]).wait()
                for j in range(_NS1):
                    pltpu.async_copy(
                        table.at[sbank[y].at[j]], bufs[y][j], sem_g[y])

            for j in range(_NS1):
                pltpu.make_async_copy(
                    table.at[sbank[x].at[j]], bufs[x][j], sem_g[x]).wait()
                pltpu.async_copy(
                    bufs[x][j], acc_sh.at[dbank[x].at[j]], sem_s, add=True)

                @pl.when(c == j % 2)
                def _():
                    pltpu.async_copy(
                        ones_v, cnt_sh.at[dbank[x].at[j]], sem_c, add=True)

            for j in range(_NS1):
                pltpu.make_async_copy(
                    bufs[x][j], acc_sh.at[dbank[x].at[j]], sem_s).wait()
            for _unused in range(_NS1 // 2):
                pltpu.make_async_copy(
                    ones_v, cnt_sh.at[dbank[x].at[0]], sem_c).wait()

            @pl.when(r + 2 < _R1)
            def _():
                pltpu.async_copy(sidx_hbm.at[s, r + 2], sbank[x], sem_i[x])
                pltpu.async_copy(didx_hbm.at[s, r + 2], dbank[x], sem_i[x])

        def body(i, carry):
            half_round(2 * i, 0)
            half_round(2 * i + 1, 1)
            return carry

        lax.fori_loop(0, _R1 // 2, body, 0)

    run(tbl_sh)

    plsc.subcore_barrier()

    rows = pl.ds(s * _RPS, _RPS)

    @pl.when(c == 0)
    def _():
        pltpu.sync_copy(acc_sh.at[rows], out_lo.at[rows])

    @pl.when(c == 1)
    def _():
        pltpu.sync_copy(acc_sh.at[rows], out_hi.at[rows])

    pltpu.sync_copy(cnt_sh.at[rows], out_cnt.at[c, rows])


@functools.partial(
    pl.kernel,
    mesh=_mesh,
    compiler_params=_sc_params,
    out_type=jax.ShapeDtypeStruct((2, _N2, _W2), jnp.float32),
    scratch_types=[
        pltpu.VMEM((_K2, _CHUNK), jnp.int32),
        pltpu.VMEM((_K2, _CHUNK), jnp.int32),
        [pltpu.VMEM((_CHUNK, _W2), jnp.float32)] * _NSLOT,
        pltpu.VMEM_SHARED((_N2, _W2), jnp.float32),
        pltpu.VMEM_SHARED((_N2, _W2), jnp.float32),
        pltpu.SemaphoreType.DMA,
        pltpu.SemaphoreType.DMA,
    ],
)
def _sc_pass2(q_hbm, src_hbm, dst_hbm, out_hbm,
              src_v, dst_v, bufs, acc_sh, tbl_sh, sem_g, sem_s):
    c = lax.axis_index("c")
    s = lax.axis_index("s")
    wid = s * 2 + c

    pltpu.sync_copy(src_hbm.at[pl.ds(wid * _K2, _K2)], src_v)
    pltpu.sync_copy(dst_hbm.at[pl.ds(wid * _K2, _K2)], dst_v)
    def stage_chunk(r0, buf):
        pltpu.sync_copy(q_hbm.at[pl.ds(r0, buf.shape[0])], buf)
        pltpu.sync_copy(buf, tbl_sh.at[pl.ds(r0, buf.shape[0])])

    @pl.when(s < 15)
    def _():
        for t in range(_RPS // _CHUNK):
            stage_chunk(s * _RPS + t * _CHUNK, bufs[1])

    @pl.when(s == 15)
    def _():
        for t in range((_N - 15 * _RPS) // _CHUNK):
            stage_chunk(15 * _RPS + t * _CHUNK, bufs[1])
        stage_chunk(_N - 16, bufs[1].at[pl.ds(0, 16)])
    _fill_rows(bufs[0], 0.0)
    _blast_stripe(bufs[0], acc_sh, s)
    plsc.subcore_barrier()

    _edge_rounds(tbl_sh, src_v, dst_v, bufs, acc_sh, sem_g, sem_s, _K2)

    plsc.subcore_barrier()
    rows = pl.ds(s * _RPS, _RPS)
    pltpu.sync_copy(acc_sh.at[rows], out_hbm.at[c, rows])


_R = 400


def _dense_body(lo_ref, hi_ref, cnt_ref, x_ref, wlo_ref, whi_ref, wrst_ref,
                b1_ref, gamma_ref, beta_ref, w2_ref, b2_ref,
                q0_ref, r_ref):
    cnt = cnt_ref[0, :, 0:1] + cnt_ref[1, :, 0:1]
    inv = 1.0 / jnp.maximum(cnt, 1.0)
    x1 = (jnp.dot(lo_ref[...] * inv, wlo_ref[...],
                  preferred_element_type=jnp.float32)
          + jnp.dot(hi_ref[...] * inv, whi_ref[...],
                    preferred_element_type=jnp.float32)
          + jnp.dot(x_ref[...], wrst_ref[...],
                    preferred_element_type=jnp.float32)
          + b1_ref[...])
    mu = jnp.mean(x1, axis=-1, keepdims=True)
    var = jnp.mean((x1 - mu) ** 2, axis=-1, keepdims=True)
    xn = (x1 - mu) * lax.rsqrt(var + 1e-5) * gamma_ref[...] + beta_ref[...]
    h = jnp.where(xn > 0, xn, jnp.exp(xn) - 1.0)
    qr = jnp.dot(h, w2_ref[...], preferred_element_type=jnp.float32) \
        + b2_ref[...]
    q0_ref[...] = qr[:, :_W2]
    r_ref[...] = qr[:, _W2:]


def _combine_body(acc2_ref, cnt_ref, r_ref, out_ref):
    cnt = cnt_ref[0, :, 0:1] + cnt_ref[1, :, 0:1]
    inv = 1.0 / jnp.maximum(cnt, 1.0)
    res = (acc2_ref[0] + acc2_ref[1]) * inv + r_ref[...]
    out_ref[...] = res[:, :_D_OUT]


def kernel(x, edge_index, Wl1, bl1, Wr1, Ws, bs, gamma, beta, Wl2, bl2, Wr2):
    src = edge_index[0]
    dst = edge_index[1]
    pad1 = _E_PAD1 - _E
    src_p = jnp.concatenate([src, jnp.zeros((pad1,), jnp.int32)])
    dst_p = jnp.concatenate([dst, jnp.full((pad1,), _N, jnp.int32)])
    src2d = src_p.reshape(32 * _K2, _CHUNK)
    dst2d = dst_p.reshape(32 * _K2, _CHUNK)
    src4 = src_p.reshape(16, _R1, _NS1, _C1)
    dst4 = dst_p.reshape(16, _R1, _NS1, _C1)

    acc_lo, acc_hi, cnt2 = _sc_pass1(x, src4, dst4)

    wl1t = Wl1.T
    wlo = wl1t[:64]
    whi = wl1t[64:128]
    wrst = (Wr1 + Ws).T
    b1 = (bl1 + bs).reshape(1, _D_H)
    g2 = gamma.reshape(1, _D_H)
    be2 = beta.reshape(1, _D_H)
    w2 = jnp.zeros((_D_H, 2 * _W2), jnp.float32)
    w2 = w2.at[:, 0:_D_OUT].set(Wl2.T).at[:, _W2:_W2 + _D_OUT].set(Wr2.T)
    b2 = jnp.zeros((1, 2 * _W2), jnp.float32)
    b2 = b2.at[0, _W2:_W2 + _D_OUT].set(bl2)

    grid = (_N // _R,)
    q0p, rp = pl.pallas_call(
        _dense_body,
        grid=grid,
        in_specs=[
            pl.BlockSpec((_R, _W1), lambda i: (i, 0)),
            pl.BlockSpec((_R, _W1), lambda i: (i, 0)),
            pl.BlockSpec((2, _R, _W2), lambda i: (0, i, 0)),
            pl.BlockSpec((_R, _D_IN), lambda i: (i, 0)),
            pl.BlockSpec((_W1, _D_H), lambda i: (0, 0)),
            pl.BlockSpec((_W1, _D_H), lambda i: (0, 0)),
            pl.BlockSpec((_D_IN, _D_H), lambda i: (0, 0)),
            pl.BlockSpec((1, _D_H), lambda i: (0, 0)),
            pl.BlockSpec((1, _D_H), lambda i: (0, 0)),
            pl.BlockSpec((1, _D_H), lambda i: (0, 0)),
            pl.BlockSpec((_D_H, 2 * _W2), lambda i: (0, 0)),
            pl.BlockSpec((1, 2 * _W2), lambda i: (0, 0)),
        ],
        out_specs=[
            pl.BlockSpec((_R, _W2), lambda i: (i, 0)),
            pl.BlockSpec((_R, _W2), lambda i: (i, 0)),
        ],
        out_shape=[
            jax.ShapeDtypeStruct((_N, _W2), jnp.float32),
            jax.ShapeDtypeStruct((_N, _W2), jnp.float32),
        ],
    )(acc_lo, acc_hi, cnt2, x, wlo, whi, wrst, b1, g2, be2, w2, b2)

    acc2 = _sc_pass2(q0p, src2d, dst2d)

    outp = pl.pallas_call(
        _combine_body,
        grid=grid,
        in_specs=[
            pl.BlockSpec((2, _R, _W2), lambda i: (0, i, 0)),
            pl.BlockSpec((2, _R, _W2), lambda i: (0, i, 0)),
            pl.BlockSpec((_R, _W2), lambda i: (i, 0)),
        ],
        out_specs=pl.BlockSpec((_R, _D_OUT), lambda i: (i, 0)),
        out_shape=jax.ShapeDtypeStruct((_N, _D_OUT), jnp.float32),
    )(acc2, cnt2, rp)

    return outp
